# baseline (speedup 1.0000x reference)
import functools

import jax
import jax.numpy as jnp
from jax import lax
from jax.experimental import pallas as pl
from jax.experimental.pallas import tpu as pltpu
from jax.experimental.pallas import tpu_sc as plsc

SEG = 16
EMB = 1024
LANES = 16
B_TOT = 4 * 8192
NC, NS = 2, 16
NW = NC * NS
BPW = B_TOT // NW
CH = 64
NCHUNK = BPW // CH

_SCALE = float(EMB) ** 0.5

_mesh = plsc.VectorSubcoreMesh(core_axis_name="c", subcore_axis_name="s")


@functools.partial(
    pl.kernel,
    out_type=(
        jax.ShapeDtypeStruct((B_TOT, EMB), jnp.float32),
        jax.ShapeDtypeStruct((SEG, EMB), jnp.float32),
    ),
    mesh=_mesh,
    scratch_types=[
        pltpu.VMEM((BPW,), jnp.int32),
        pltpu.VMEM((SEG, EMB), jnp.float32),
        pltpu.VMEM((CH, EMB), jnp.float32),
        pltpu.SemaphoreType.DMA,
    ],
)
def _emb_kernel(ids_hbm, w_hbm, out_hbm, scaled_hbm, idx_v, table_v, rows_v, sem):
    wid = lax.axis_index("s") * NC + lax.axis_index("c")
    base = wid * BPW

    pltpu.sync_copy(ids_hbm.at[pl.ds(base, BPW)], idx_v)

    pltpu.sync_copy(w_hbm, table_v)
    for r in range(SEG):
        for j in range(EMB // LANES):
            table_v[r, pl.ds(j * LANES, LANES)] = (
                table_v[r, pl.ds(j * LANES, LANES)] * _SCALE
            )
    pltpu.sync_copy(table_v, scaled_hbm)

    def chunk_body(ci, carry):
        row0 = ci * CH
        pltpu.async_copy(
            scaled_hbm.at[idx_v.at[pl.ds(row0, CH)]], rows_v, sem
        ).wait()
        pltpu.sync_copy(rows_v, out_hbm.at[pl.ds(base + row0, CH)])
        return carry

    lax.fori_loop(0, NCHUNK, chunk_body, 0)


def kernel(segment_ids, weight):
    ids_flat = segment_ids.reshape(-1).astype(jnp.int32)
    out, _scaled = _emb_kernel(ids_flat, weight)
    return out.reshape(segment_ids.shape + (EMB,))
